# Initial kernel scaffold; baseline (speedup 1.0000x reference)
import jax
import jax.numpy as jnp
from jax.experimental import pallas as pl


def kernel(velocity_length, velocity_theta, length, width, a_type, position, heading, lane_position, lane_heading, l_embs, mode_tokens, aW1, ab1, aW2, ab2, tW1, tb1, tW2, tb2, lW1, lb1, lW2, lb2, qA, kA, vA, oA, qL, kL, vL, oL, pW1, pb1, pW2, pb2, agent_batch, lane_batch):
    raise NotImplementedError("write your pallas kernel here")


# speedup vs baseline: 5.3721x; 5.3721x over previous
import numpy as np
import jax
import jax.numpy as jnp
from jax.experimental import pallas as pl

A = 64; H = 50; MODES = 6; D = 128; F = 60; L = 512
NHEADS = 8; DH = D // NHEADS
DURATION = 10; W = DURATION + 1
L2A_RADIUS = 50.0
N_M = A * H * MODES
F32 = jnp.float32
_SCALE = 1.0 / (DH ** 0.5)
_DN_MINOR = (((1,), (1,)), ((), ()))


def _wrap(a):
    two_pi = 2.0 * np.pi
    w = a + np.pi
    w = w - two_pi * jnp.floor(w / two_pi)
    return w - np.pi


def _mm(x, w):
    return jnp.dot(x, w, preferred_element_type=F32)


def _iota(shape, dim):
    return jax.lax.broadcasted_iota(jnp.int32, shape, dim)


def _head_mats(q):
    mh = MODES * NHEADS
    et = (_iota((mh, MODES), 0) // NHEADS == _iota((mh, MODES), 1)).astype(F32)
    hm = (_iota((mh, D), 1) // DH == _iota((mh, D), 0) % NHEADS).astype(F32)
    qk = _mm(et, q) * hm
    x8 = (_iota((NHEADS, D), 1) // DH == _iota((NHEADS, D), 0)).astype(F32)
    return qk, x8


def _pre_kernel(l_embs_ref, kL_ref, vL_ref, k_out_ref, v_out_ref):
    le = l_embs_ref[...]
    k_out_ref[...] = _mm(le, kL_ref[...])
    v_out_ref[...] = _mm(le, vL_ref[...])


def _agent_kernel(a_in_ref, pos_ref, head_ref, mode_tokens_ref,
                  aW1_ref, ab1_ref, aW2_ref, ab2_ref,
                  tW1_ref, tb1_ref, tW2_ref, tb2_ref,
                  qA_ref, kA_ref, vA_ref, oA_ref,
                  m_out_ref):
    a_in = a_in_ref[0]
    pos = pos_ref[0]
    headc = head_ref[0]
    mode_tokens = mode_tokens_ref[...]

    x1 = jnp.maximum(_mm(a_in, aW1_ref[...]) + ab1_ref[...], 0.0)
    a_embs = _mm(x1, aW2_ref[...]) + ab2_ref[...]
    k = _mm(a_embs, kA_ref[...])
    v = _mm(a_embs, vA_ref[...])
    q = _mm(mode_tokens, qA_ref[...])
    qk, x8 = _head_mats(q)

    row = _iota((H, H), 0)
    colm = _iota((H, H), 1)
    hrow = _iota((H, 1), 0)
    c = jnp.cos(headc); s = jnp.sin(headc)
    px = pos[:, 0:1]; py = pos[:, 1:2]

    scs = []; ves = []; valids = []
    for w in range(W):
        shift = DURATION - w
        ohw = (colm == row - shift).astype(F32)
        validw = hrow >= shift
        k_w = _mm(ohw, k); v_w = _mm(ohw, v)
        pos_w = _mm(ohw, pos)
        head_w = _mm(ohw, headc)
        dx = pos_w[:, 0:1] - px
        dy = pos_w[:, 1:2] - py
        xr = dx * c + dy * s
        yr = -dx * s + dy * c
        ln = jnp.sqrt(xr * xr + yr * yr + 1e-12)
        th = jnp.arctan2(yr, xr)
        hd = _wrap(head_w - headc)
        intv = jnp.full((H, 1), float(w - DURATION), dtype=F32)
        feat = jnp.concatenate([ln, th, hd, intv], axis=1)
        f1 = jnp.maximum(_mm(feat, tW1_ref[...]) + tb1_ref[...], 0.0)
        attr = _mm(f1, tW2_ref[...]) + tb2_ref[...]
        ke = k_w + attr
        ve = v_w + attr
        sc = jax.lax.dot_general(ke, qk, _DN_MINOR,
                                 preferred_element_type=F32) * _SCALE
        sc = jnp.where(validw, sc, -jnp.inf)
        scs.append(sc); ves.append(ve); valids.append(validw)

    mx = scs[0]
    for w in range(1, W):
        mx = jnp.maximum(mx, scs[w])
    es = []
    den = jnp.zeros_like(mx)
    for w in range(W):
        e = jnp.where(valids[w], jnp.exp(scs[w] - mx), 0.0)
        es.append(e)
        den = den + e
    den = den + 1e-9

    for mo in range(MODES):
        agg = jnp.zeros((H, D), dtype=F32)
        lo = mo * NHEADS
        for w in range(W):
            al = es[w][:, lo:lo + NHEADS] / den[:, lo:lo + NHEADS]
            agg = agg + _mm(al, x8) * ves[w]
        m_out = mode_tokens[mo] + _mm(agg, oA_ref[...])
        m_out_ref[0, mo * H:(mo + 1) * H, :] = m_out


def _lane_kernel(m_ref, pos_ref, head_ref, same_ref,
                 lane_pos_ref, lane_head_ref, k_lane_ref, v_lane_ref,
                 lW1_ref, lb1_ref, lW2_ref, lb2_ref,
                 qL_ref, oL_ref, pW1_ref, pb1_ref, pW2_ref, pb2_ref,
                 out_ref):
    m = m_ref[0, 0]
    px = pos_ref[0, 0, 0, 0]
    py = pos_ref[0, 0, 0, 1]
    head = head_ref[0, 0, 0, 0]
    same = same_ref[0]
    lane_pos = lane_pos_ref[...]
    lane_head = lane_head_ref[...]

    rx = lane_pos[:, 0:1] - px
    ry = lane_pos[:, 1:2] - py
    c = jnp.cos(head); s = jnp.sin(head)
    xr = rx * c + ry * s
    yr = -rx * s + ry * c
    d2 = rx * rx + ry * ry
    ln = jnp.sqrt(xr * xr + yr * yr + 1e-12)
    th = jnp.arctan2(yr, xr)
    hd = _wrap(lane_head - head)
    feat = jnp.concatenate([ln, th, hd], axis=1)

    f1 = jnp.maximum(_mm(feat, lW1_ref[...]) + lb1_ref[...], 0.0)
    attr = _mm(f1, lW2_ref[...]) + lb2_ref[...]
    kp = k_lane_ref[...] + attr
    vp = v_lane_ref[...] + attr

    q2 = _mm(m, qL_ref[...])
    qk, x8 = _head_mats(q2)

    mk = (same > 0.5) & (jnp.sqrt(d2) < L2A_RADIUS)
    sc = jax.lax.dot_general(kp, qk, _DN_MINOR,
                             preferred_element_type=F32) * _SCALE
    sc = jnp.where(mk, sc, -jnp.inf)
    mx = sc.max(axis=0, keepdims=True)
    mx = jnp.where(mx > -3e38, mx, 0.0)
    e = jnp.where(mk, jnp.exp(sc - mx), 0.0)
    den = e.sum(axis=0, keepdims=True) + 1e-9
    alpha = e / den

    rows = []
    for mo in range(MODES):
        aexp = _mm(alpha[:, mo * NHEADS:(mo + 1) * NHEADS], x8)
        rows.append((aexp * vp).sum(axis=0, keepdims=True))
    agg_all = jnp.concatenate(rows, axis=0)
    m_out = m + _mm(agg_all, oL_ref[...])

    p1 = jnp.maximum(_mm(m_out, pW1_ref[...]) + pb1_ref[...], 0.0)
    traj = _mm(p1, pW2_ref[...]) + pb2_ref[...]
    out_ref[0, 0] = traj


def _full(shape):
    nd = len(shape)
    return pl.BlockSpec(shape, lambda *args: (0,) * nd)


def kernel(velocity_length, velocity_theta, length, width, a_type, position,
           heading, lane_position, lane_heading, l_embs, mode_tokens,
           aW1, ab1, aW2, ab2, tW1, tb1, tW2, tb2, lW1, lb1, lW2, lb2,
           qA, kA, vA, oA, qL, kL, vL, oL, pW1, pb1, pW2, pb2,
           agent_batch, lane_batch):
    a_in = jnp.stack([
        velocity_length, velocity_theta,
        jnp.broadcast_to(length[:, None], (A, H)),
        jnp.broadcast_to(width[:, None], (A, H)),
        jnp.broadcast_to(a_type[:, None], (A, H)),
    ], axis=-1)
    same = (agent_batch[:, None] == lane_batch[None, :]).astype(F32)
    same3 = same.reshape(A, L, 1)
    head3 = heading.reshape(A, H, 1)
    pos4 = position.reshape(A, H, 1, 2)
    head4 = heading.reshape(A, H, 1, 1)
    lane_head2 = lane_heading.reshape(L, 1)

    k_lane, v_lane = pl.pallas_call(
        _pre_kernel,
        out_shape=[jax.ShapeDtypeStruct((L, D), F32),
                   jax.ShapeDtypeStruct((L, D), F32)],
    )(l_embs, kL, vL)

    m_flat = pl.pallas_call(
        _agent_kernel,
        grid=(A,),
        in_specs=[
            pl.BlockSpec((1, H, 5), lambda a: (a, 0, 0)),
            pl.BlockSpec((1, H, 2), lambda a: (a, 0, 0)),
            pl.BlockSpec((1, H, 1), lambda a: (a, 0, 0)),
            _full((MODES, D)),
            _full((5, D)), _full((D,)), _full((D, D)), _full((D,)),
            _full((4, D)), _full((D,)), _full((D, D)), _full((D,)),
            _full((D, D)), _full((D, D)), _full((D, D)), _full((D, D)),
        ],
        out_specs=pl.BlockSpec((1, MODES * H, D), lambda a: (a, 0, 0)),
        out_shape=jax.ShapeDtypeStruct((A, MODES * H, D), F32),
    )(a_in, position, head3, mode_tokens,
      aW1, ab1, aW2, ab2, tW1, tb1, tW2, tb2, qA, kA, vA, oA)

    m = m_flat.reshape(A, MODES, H, D).transpose(0, 2, 1, 3)

    traj = pl.pallas_call(
        _lane_kernel,
        grid=(A, H),
        in_specs=[
            pl.BlockSpec((1, 1, MODES, D), lambda a, h: (a, h, 0, 0)),
            pl.BlockSpec((1, 1, 1, 2), lambda a, h: (a, h, 0, 0)),
            pl.BlockSpec((1, 1, 1, 1), lambda a, h: (a, h, 0, 0)),
            pl.BlockSpec((1, L, 1), lambda a, h: (a, 0, 0)),
            _full((L, 2)), _full((L, 1)), _full((L, D)), _full((L, D)),
            _full((3, D)), _full((D,)), _full((D, D)), _full((D,)),
            _full((D, D)), _full((D, D)),
            _full((D, D)), _full((D,)), _full((D, 2 * F)), _full((2 * F,)),
        ],
        out_specs=pl.BlockSpec((1, 1, MODES, 2 * F), lambda a, h: (a, h, 0, 0)),
        out_shape=jax.ShapeDtypeStruct((A, H, MODES, 2 * F), F32),
    )(m, pos4, head4, same3, lane_position, lane_head2,
      k_lane, v_lane, lW1, lb1, lW2, lb2, qL, oL, pW1, pb1, pW2, pb2)

    return traj.reshape(N_M, 2 * F)
